# Initial kernel scaffold; baseline (speedup 1.0000x reference)
import jax
import jax.numpy as jnp
from jax.experimental import pallas as pl


def kernel(x, edge_index, hist, replica_mask, W_self1, W_neigh1, b1, W_self2, W_neigh2, b2, gate):
    raise NotImplementedError("write your pallas kernel here")


# speedup vs baseline: 4.8905x; 4.8905x over previous
import functools

import jax
import jax.numpy as jnp
from jax import lax
from jax.experimental import pallas as pl
from jax.experimental.pallas import tpu as pltpu
from jax.experimental.pallas import tpu_sc as plsc

N_NODES = 10000
N_EDGES = 160000
D_IN = 256
D_HID = 256
D_OUT = 64

NC = 2
NS = 16
L = 16
DH = D_IN // 2
CH = 128
NCHUNKS = N_EDGES // CH
R_MAIN = 624
TAIL = 16
TAIL_OFF = N_NODES - TAIL


def _fill(ref, nrows, ncols, value):
    vec = jnp.full((L,), value, jnp.float32)

    def body(i, carry):
        for j in range(ncols // L):
            ref[i, pl.ds(j * L, L)] = vec
        return carry

    lax.fori_loop(0, nrows, body, 0)


def _zero_span(tmpl, dst, r0):
    for j in range(4):
        pltpu.sync_copy(tmpl, dst.at[pl.ds(r0 + j * CH, CH)])
    pltpu.sync_copy(tmpl.at[pl.ds(0, R_MAIN - 4 * CH)],
                    dst.at[pl.ds(r0 + 4 * CH, R_MAIN - 4 * CH)])


def _mesh():
    return plsc.VectorSubcoreMesh(core_axis_name="c", subcore_axis_name="s",
                                  num_cores=NC, num_subcores=NS)


@functools.cache
def _build_sc_agg1():
    @functools.partial(
        pl.kernel,
        out_type=[
            jax.ShapeDtypeStruct((NC, N_NODES, DH), jnp.float32),
            jax.ShapeDtypeStruct((NC, N_NODES, DH), jnp.float32),
        ],
        mesh=_mesh(),
        scratch_types=[
            pltpu.VMEM_SHARED((N_NODES, DH), jnp.float32),
            pltpu.VMEM((CH,), jnp.int32),
            pltpu.VMEM((1, CH), jnp.int32),
            pltpu.VMEM((CH, DH), jnp.float32),
            pltpu.SemaphoreType.DMA,
        ],
    )
    def sc_agg1(xcat, src, dst, agg_out, deg_out, acc, sbuf, dbuf, rows, sem):
        c = lax.axis_index("c")
        s = lax.axis_index("s")
        wid = s * NC + c
        r0 = s * R_MAIN

        def zero_acc():
            _fill(rows, CH, DH, 0.0)
            _zero_span(rows, acc, r0)

            @pl.when(s == 0)
            def _():
                pltpu.sync_copy(rows.at[pl.ds(0, TAIL)],
                                acc.at[pl.ds(TAIL_OFF, TAIL)])

        def dump_acc(out):
            pltpu.sync_copy(acc.at[pl.ds(r0, R_MAIN)],
                            out.at[c, pl.ds(r0, R_MAIN)])

            @pl.when(s == 0)
            def _():
                pltpu.sync_copy(acc.at[pl.ds(TAIL_OFF, TAIL)],
                                out.at[c, pl.ds(TAIL_OFF, TAIL)])

        zero_acc()
        _fill(rows, CH, DH, 1.0)
        plsc.subcore_barrier()

        def deg_body(k, carry):
            chunk = k * (NC * NS) + wid

            @pl.when(chunk < NCHUNKS)
            def _():
                pltpu.sync_copy(dst.at[pl.ds(chunk * CH, CH)], dbuf.at[0])
                pltpu.sync_copy(rows, acc.at[dbuf.at[0]], add=True)

            return carry

        lax.fori_loop(0, (NCHUNKS + NC * NS - 1) // (NC * NS), deg_body, 0)

        plsc.subcore_barrier()
        dump_acc(deg_out)
        zero_acc()
        plsc.subcore_barrier()

        base = c * N_NODES

        def chunk_body(k, carry):
            chunk = k * NS + s

            @pl.when(chunk < NCHUNKS)
            def _():
                off = chunk * CH
                pltpu.sync_copy(src.at[pl.ds(off, CH)], sbuf)
                pltpu.sync_copy(dst.at[pl.ds(off, CH)], dbuf.at[0])
                for j in range(CH // L):
                    sl = pl.ds(j * L, L)
                    sbuf[sl] = sbuf[sl] + base
                pltpu.async_copy(xcat.at[sbuf], rows, sem).wait()
                pltpu.sync_copy(rows, acc.at[dbuf.at[0]], add=True)

            return carry

        lax.fori_loop(0, (NCHUNKS + NS - 1) // NS, chunk_body, 0)

        plsc.subcore_barrier()
        dump_acc(agg_out)

    return sc_agg1


@functools.cache
def _build_sc_agg2():
    @functools.partial(
        pl.kernel,
        out_type=jax.ShapeDtypeStruct((NC, N_NODES, DH), jnp.float32),
        mesh=_mesh(),
        scratch_types=[
            pltpu.VMEM_SHARED((N_NODES, DH), jnp.float32),
            pltpu.VMEM((CH,), jnp.int32),
            pltpu.VMEM((1, CH), jnp.int32),
            pltpu.VMEM((CH, DH), jnp.float32),
            pltpu.SemaphoreType.DMA,
        ],
    )
    def sc_agg2(sp, src, dst, agg_out, acc, sbuf, dbuf, rows, sem):
        c = lax.axis_index("c")
        s = lax.axis_index("s")
        wid = s * NC + c
        r0 = s * R_MAIN

        _fill(rows, CH, DH, 0.0)
        _zero_span(rows, acc, r0)

        @pl.when(s == 0)
        def _():
            pltpu.sync_copy(rows.at[pl.ds(0, TAIL)], acc.at[pl.ds(TAIL_OFF, TAIL)])

        plsc.subcore_barrier()

        def chunk_body(k, carry):
            chunk = k * (NC * NS) + wid

            @pl.when(chunk < NCHUNKS)
            def _():
                off = chunk * CH
                pltpu.sync_copy(src.at[pl.ds(off, CH)], sbuf)
                pltpu.sync_copy(dst.at[pl.ds(off, CH)], dbuf.at[0])
                pltpu.async_copy(sp.at[sbuf], rows, sem).wait()
                pltpu.sync_copy(rows, acc.at[dbuf.at[0]], add=True)

            return carry

        lax.fori_loop(0, (NCHUNKS + NC * NS - 1) // (NC * NS), chunk_body, 0)

        plsc.subcore_barrier()

        pltpu.sync_copy(acc.at[pl.ds(r0, R_MAIN)],
                        agg_out.at[c, pl.ds(r0, R_MAIN)])

        @pl.when(s == 0)
        def _():
            pltpu.sync_copy(acc.at[pl.ds(TAIL_OFF, TAIL)],
                            agg_out.at[c, pl.ds(TAIL_OFF, TAIL)])

    return sc_agg2


BLK = 1000


def _tc_layer1_body(x_ref, agg_ref, deg_ref, w1_ref, b1_ref, w2_ref,
                    h1_ref, sp_ref):
    deg = deg_ref[0, :, 0:1] + deg_ref[1, :, 0:1]
    inv = 1.0 / jnp.maximum(deg, 1.0)
    mean = jnp.concatenate([agg_ref[0], agg_ref[1]], axis=1) * inv
    xm = jnp.concatenate([x_ref[...], mean], axis=1)
    h1 = jnp.dot(xm, w1_ref[...], preferred_element_type=jnp.float32) + b1_ref[...]
    h1_ref[...] = h1
    hb = jnp.maximum(h1, 0.0)
    sp_ref[...] = jnp.dot(hb, w2_ref[...], preferred_element_type=jnp.float32)


_tc_layer1 = pl.pallas_call(
    _tc_layer1_body,
    grid=(N_NODES // BLK,),
    in_specs=[
        pl.BlockSpec((BLK, D_IN), lambda i: (i, 0)),
        pl.BlockSpec((NC, BLK, DH), lambda i: (0, i, 0)),
        pl.BlockSpec((NC, BLK, DH), lambda i: (0, i, 0)),
        pl.BlockSpec((2 * D_IN, D_HID), lambda i: (0, 0)),
        pl.BlockSpec((1, D_HID), lambda i: (0, 0)),
        pl.BlockSpec((D_HID, 2 * D_OUT), lambda i: (0, 0)),
    ],
    out_specs=[
        pl.BlockSpec((BLK, D_HID), lambda i: (i, 0)),
        pl.BlockSpec((BLK, 2 * D_OUT), lambda i: (i, 0)),
    ],
    out_shape=[
        jax.ShapeDtypeStruct((N_NODES, D_HID), jnp.float32),
        jax.ShapeDtypeStruct((N_NODES, 2 * D_OUT), jnp.float32),
    ],
)


def _tc_final_body(sp_ref, agg2_ref, deg_ref, b2_ref, out_ref):
    deg = deg_ref[0, :, 0:1] + deg_ref[1, :, 0:1]
    inv = 1.0 / jnp.maximum(deg, 1.0)
    aggp = agg2_ref[0, :, D_OUT:] + agg2_ref[1, :, D_OUT:]
    out_ref[...] = sp_ref[:, :D_OUT] + aggp * inv + b2_ref[...]


_tc_final = pl.pallas_call(
    _tc_final_body,
    grid=(N_NODES // BLK,),
    in_specs=[
        pl.BlockSpec((BLK, 2 * D_OUT), lambda i: (i, 0)),
        pl.BlockSpec((NC, BLK, DH), lambda i: (0, i, 0)),
        pl.BlockSpec((NC, BLK, DH), lambda i: (0, i, 0)),
        pl.BlockSpec((1, D_OUT), lambda i: (0, 0)),
    ],
    out_specs=pl.BlockSpec((BLK, D_OUT), lambda i: (i, 0)),
    out_shape=jax.ShapeDtypeStruct((N_NODES, D_OUT), jnp.float32),
)


def kernel(x, edge_index, hist, replica_mask,
           W_self1, W_neigh1, b1, W_self2, W_neigh2, b2, gate):
    src = edge_index[0]
    dst = edge_index[1]
    xcat = x.reshape(N_NODES, NC, DH).transpose(1, 0, 2).reshape(NC * N_NODES, DH)
    agg1, degtab = _build_sc_agg1()(xcat, src, dst)
    W1 = jnp.concatenate([W_self1, W_neigh1], axis=0)
    W2 = jnp.concatenate([W_self2, W_neigh2], axis=1)
    h1, sp = _tc_layer1(x, agg1, degtab, W1, b1.reshape(1, -1), W2)
    agg2 = _build_sc_agg2()(sp, src, dst)
    h2 = _tc_final(sp, agg2, degtab, b2.reshape(1, -1))
    return h2, h1
